# baseline (device time: 28425 ns/iter reference)
import jax
import jax.numpy as jnp
from jax import lax
from jax.experimental import pallas as pl
from jax.experimental.pallas import tpu as pltpu


def kernel(x, router, W1, W2):
    t, d = x.shape
    e_loc = W1.shape[0]
    f32 = jnp.float32
    bf16 = jnp.bfloat16

    def body(x_ref, r_ref, w1_ref, w2_ref, out_ref,
             x_send, xr_comm, r_comm, wt_send, wt_comm,
             part_send, part_comm, send_sems, recv_sems):
        my_x = lax.axis_index("x")
        my_y = lax.axis_index("y")
        my_z = lax.axis_index("z")
        peer = (my_x, 1 - my_y, my_z)
        i_am_lo = my_y == 0

        barrier = pltpu.get_barrier_semaphore()
        pl.semaphore_signal(barrier, inc=1, device_id=peer,
                            device_id_type=pl.DeviceIdType.MESH)
        pl.semaphore_wait(barrier, 1)

        x_send[...] = x_ref[...].astype(bf16)
        rdma_x = pltpu.make_async_remote_copy(
            src_ref=x_send, dst_ref=xr_comm,
            send_sem=send_sems.at[0], recv_sem=recv_sems.at[0],
            device_id=peer, device_id_type=pl.DeviceIdType.MESH)
        rdma_x.start()
        rdma_r = pltpu.make_async_remote_copy(
            src_ref=r_ref, dst_ref=r_comm,
            send_sem=send_sems.at[1], recv_sem=recv_sems.at[1],
            device_id=peer, device_id_type=pl.DeviceIdType.MESH)
        rdma_r.start()
        rdma_r.wait()

        g_mine = jnp.dot(x_ref[...], r_ref[...],
                         precision=lax.Precision.HIGHEST)
        g_peer = jnp.dot(x_ref[...], r_comm[...],
                         precision=lax.Precision.HIGHEST)
        g = jnp.where(i_am_lo,
                      jnp.concatenate([g_mine, g_peer], axis=1),
                      jnp.concatenate([g_peer, g_mine], axis=1))

        m1 = jnp.max(g, axis=1, keepdims=True)
        is_top1 = g == m1
        g_rest = jnp.where(is_top1, -jnp.inf, g)
        m2 = jnp.max(g_rest, axis=1, keepdims=True)
        sel = is_top1 | (g_rest == m2)
        ex = jnp.where(sel, jnp.exp(g - m1), 0.0)
        w = ex / jnp.sum(ex, axis=1, keepdims=True)

        w_mine = jnp.where(i_am_lo, w[:, :e_loc], w[:, e_loc:])
        wt_send[...] = jnp.where(i_am_lo, w[:, e_loc:], w[:, :e_loc])
        rdma_w = pltpu.make_async_remote_copy(
            src_ref=wt_send, dst_ref=wt_comm,
            send_sem=send_sems.at[2], recv_sem=recv_sems.at[2],
            device_id=peer, device_id_type=pl.DeviceIdType.MESH)
        rdma_w.start()
        rdma_x.wait()
        rdma_w.wait()

        x_all = jnp.concatenate([x_send[...], xr_comm[...]], axis=0)
        w_all = jnp.concatenate([w_mine, wt_comm[...]], axis=0)
        acc = jnp.zeros((2 * t, d), f32)
        for e in range(e_loc):
            h = jnp.maximum(
                jnp.dot(x_all, w1_ref[e].astype(bf16),
                        preferred_element_type=f32), 0.0)
            y_e = jnp.dot(h.astype(bf16), w2_ref[e].astype(bf16),
                          preferred_element_type=f32)
            acc += y_e * w_all[:, e:e + 1]

        part_send[...] = acc[t:].astype(bf16)
        rdma_p = pltpu.make_async_remote_copy(
            src_ref=part_send, dst_ref=part_comm,
            send_sem=send_sems.at[3], recv_sem=recv_sems.at[3],
            device_id=peer, device_id_type=pl.DeviceIdType.MESH)
        rdma_p.start()
        rdma_p.wait()
        out_ref[...] = acc[:t] + part_comm[...].astype(f32)

    return pl.pallas_call(
        body,
        out_shape=jax.ShapeDtypeStruct((t, d), f32),
        in_specs=[pl.BlockSpec(memory_space=pltpu.VMEM)] * 4,
        out_specs=pl.BlockSpec(memory_space=pltpu.VMEM),
        scratch_shapes=[
            pltpu.VMEM((t, d), bf16),
            pltpu.VMEM((t, d), bf16),
            pltpu.VMEM((d, e_loc), f32),
            pltpu.VMEM((t, e_loc), f32),
            pltpu.VMEM((t, e_loc), f32),
            pltpu.VMEM((t, d), bf16),
            pltpu.VMEM((t, d), bf16),
            pltpu.SemaphoreType.DMA((4,)),
            pltpu.SemaphoreType.DMA((4,)),
        ],
        compiler_params=pltpu.CompilerParams(collective_id=0),
    )(x, router, W1, W2)


# device time: 27327 ns/iter; 1.0402x vs baseline; 1.0402x over previous
import jax
import jax.numpy as jnp
from jax import lax
from jax.experimental import pallas as pl
from jax.experimental.pallas import tpu as pltpu


def kernel(x, router, W1, W2):
    t, d = x.shape
    e_loc = W1.shape[0]
    f32 = jnp.float32
    bf16 = jnp.bfloat16

    def body(x_ref, r_ref, w1_ref, w2_ref, out_ref,
             x_send, xr_comm, r_comm, wt_send, wt_comm,
             part_send, part_comm, send_sems, recv_sems):
        my_x = lax.axis_index("x")
        my_y = lax.axis_index("y")
        my_z = lax.axis_index("z")
        peer = (my_x, 1 - my_y, my_z)
        i_am_lo = my_y == 0

        barrier = pltpu.get_barrier_semaphore()
        pl.semaphore_signal(barrier, inc=1, device_id=peer,
                            device_id_type=pl.DeviceIdType.MESH)
        pl.semaphore_wait(barrier, 1)

        x_send[...] = x_ref[...].astype(bf16)
        rdma_x = pltpu.make_async_remote_copy(
            src_ref=x_send, dst_ref=xr_comm,
            send_sem=send_sems.at[0], recv_sem=recv_sems.at[0],
            device_id=peer, device_id_type=pl.DeviceIdType.MESH)
        rdma_x.start()
        rdma_r = pltpu.make_async_remote_copy(
            src_ref=r_ref, dst_ref=r_comm,
            send_sem=send_sems.at[1], recv_sem=recv_sems.at[1],
            device_id=peer, device_id_type=pl.DeviceIdType.MESH)
        rdma_r.start()

        w1b = [w1_ref[e].astype(bf16) for e in range(e_loc)]
        w2b = [w2_ref[e].astype(bf16) for e in range(e_loc)]
        rdma_r.wait()

        g_mine = jnp.dot(x_ref[...], r_ref[...],
                         precision=lax.Precision.HIGHEST)
        g_peer = jnp.dot(x_ref[...], r_comm[...],
                         precision=lax.Precision.HIGHEST)
        g = jnp.where(i_am_lo,
                      jnp.concatenate([g_mine, g_peer], axis=1),
                      jnp.concatenate([g_peer, g_mine], axis=1))

        m1 = jnp.max(g, axis=1, keepdims=True)
        is_top1 = g == m1
        g_rest = jnp.where(is_top1, -jnp.inf, g)
        m2 = jnp.max(g_rest, axis=1, keepdims=True)
        sel = is_top1 | (g_rest == m2)
        ex = jnp.where(sel, jnp.exp(g - m1), 0.0)
        w = ex / jnp.sum(ex, axis=1, keepdims=True)

        w_mine = jnp.where(i_am_lo, w[:, :e_loc], w[:, e_loc:])
        wt_send[...] = jnp.where(i_am_lo, w[:, e_loc:], w[:, :e_loc])
        rdma_w = pltpu.make_async_remote_copy(
            src_ref=wt_send, dst_ref=wt_comm,
            send_sem=send_sems.at[2], recv_sem=recv_sems.at[2],
            device_id=peer, device_id_type=pl.DeviceIdType.MESH)
        rdma_w.start()

        def ffn(xv, wv):
            acc = jnp.zeros((t, d), f32)
            for e in range(e_loc):
                h = jnp.maximum(
                    jnp.dot(xv, w1b[e], preferred_element_type=f32), 0)
                acc += jnp.dot(h.astype(bf16), w2b[e],
                               preferred_element_type=f32) * wv[:, e:e + 1]
            return acc

        acc_mine = ffn(x_send[...], w_mine)
        rdma_x.wait()
        rdma_w.wait()

        acc_peer = ffn(xr_comm[...], wt_comm[...])
        part_send[...] = acc_peer.astype(bf16)
        rdma_p = pltpu.make_async_remote_copy(
            src_ref=part_send, dst_ref=part_comm,
            send_sem=send_sems.at[3], recv_sem=recv_sems.at[3],
            device_id=peer, device_id_type=pl.DeviceIdType.MESH)
        rdma_p.start()
        rdma_p.wait()
        out_ref[...] = acc_mine + part_comm[...].astype(f32)

    return pl.pallas_call(
        body,
        out_shape=jax.ShapeDtypeStruct((t, d), f32),
        in_specs=[pl.BlockSpec(memory_space=pltpu.VMEM)] * 4,
        out_specs=pl.BlockSpec(memory_space=pltpu.VMEM),
        scratch_shapes=[
            pltpu.VMEM((t, d), bf16),
            pltpu.VMEM((t, d), bf16),
            pltpu.VMEM((d, e_loc), f32),
            pltpu.VMEM((t, e_loc), f32),
            pltpu.VMEM((t, e_loc), f32),
            pltpu.VMEM((t, d), bf16),
            pltpu.VMEM((t, d), bf16),
            pltpu.SemaphoreType.DMA((4,)),
            pltpu.SemaphoreType.DMA((4,)),
        ],
        compiler_params=pltpu.CompilerParams(collective_id=0),
    )(x, router, W1, W2)
